# Initial kernel scaffold; baseline (speedup 1.0000x reference)
import jax
import jax.numpy as jnp
from jax.experimental import pallas as pl


def kernel(x, edge_index, user_embedding, item_embedding, W_gu, b_gu, W_gi, b_gi, W1, b1, W2, b2, W3, b3, user_bias, item_bias):
    raise NotImplementedError("write your pallas kernel here")


# speedup vs baseline: 6.0888x; 6.0888x over previous
import functools

import jax
import jax.numpy as jnp
from jax import lax
from jax.experimental import pallas as pl
from jax.experimental.pallas import tpu as pltpu
from jax.experimental.pallas import tpu_sc as plsc

N_USER = 50000
N_ITEM = 50000
N_EDGES = 800000
D = 64
B = 16384
L = 16

TRASH = B
CH = 128
R = 16512
RCH = R // CH
NSUB = 16
Q_PER_TILE = B // 32
EPT = N_EDGES // NSUB
BLKE = 400
NBLK = EPT // BLKE
GRP = BLKE // L
STCAP = 640
NSTG = (EPT // CH + 2) * CH

_mesh = plsc.VectorSubcoreMesh(core_axis_name="c", subcore_axis_name="s")
_sc_params = pltpu.CompilerParams(needs_layout_passes=False,
                                  use_tc_tiling_on_sc=False)


def _iota16():
    return lax.iota(jnp.int32, L)


NPAD = 50016
NPACK = NPAD // 2


@functools.partial(
    pl.kernel,
    out_type=(
        jax.ShapeDtypeStruct((NPACK,), jnp.int32),
        jax.ShapeDtypeStruct((NPACK,), jnp.int32),
        jax.ShapeDtypeStruct((B,), jnp.int32),
        jax.ShapeDtypeStruct((B,), jnp.int32),
    ),
    mesh=_mesh,
    compiler_params=_sc_params,
    scratch_types=[
        pltpu.VMEM((2 * B,), jnp.int32),
        pltpu.VMEM((NPAD,), jnp.int32),
        pltpu.VMEM((B,), jnp.int32),
        pltpu.VMEM((B,), jnp.int32),
    ],
)
def _slot_kernel(x_hbm, slotu, sloti, repu, repi, xv, table, ids, rep):
    c = lax.axis_index("c")
    s = lax.axis_index("s")

    @pl.when(s == 0)
    def _():
        pltpu.sync_copy(x_hbm, xv)

        def init_body(i, carry):
            table[pl.ds(i * L, L)] = jnp.full((L,), TRASH, jnp.int32)
            return carry
        lax.fori_loop(0, NPAD // L, init_body, 0)

        def extract_body(q, carry):
            qv = q * L + _iota16()
            v = plsc.load_gather(xv, [qv * 2 + c])
            ids[pl.ds(q * L, L)] = v
            plsc.store_scatter(table, [v], qv)
            return carry
        lax.fori_loop(0, B // L, extract_body, 0)

        def rep_body(q, carry):
            kv = ids[pl.ds(q * L, L)]
            rep[pl.ds(q * L, L)] = plsc.load_gather(table, [kv])
            return carry
        lax.fori_loop(0, B // L, rep_body, 0)

        def pack_body(j, carry):
            jv = j * L + _iota16()
            lo = plsc.load_gather(table, [jv * 2])
            hi = plsc.load_gather(table, [jv * 2 + 1])
            xv[pl.ds(j * L, L)] = lo | (hi << 16)
            return carry
        lax.fori_loop(0, NPACK // L, pack_body, 0)

        @pl.when(c == 0)
        def _():
            pltpu.sync_copy(xv.at[pl.ds(0, NPACK)], slotu)
            pltpu.sync_copy(rep, repu)

        @pl.when(c == 1)
        def _():
            pltpu.sync_copy(xv.at[pl.ds(0, NPACK)], sloti)
            pltpu.sync_copy(rep, repi)


@functools.partial(
    pl.kernel,
    out_type=(
        jax.ShapeDtypeStruct((R, D), jnp.float32),
        jax.ShapeDtypeStruct((R, D), jnp.float32),
        jax.ShapeDtypeStruct((R, L), jnp.float32),
        jax.ShapeDtypeStruct((R, L), jnp.float32),
    ),
    mesh=_mesh,
    compiler_params=_sc_params,
    scratch_types=[
        pltpu.VMEM((NPACK,), jnp.int32),
        pltpu.VMEM((CH,), jnp.int32),
        pltpu.VMEM((CH,), jnp.int32),
        pltpu.VMEM((CH,), jnp.int32),
        pltpu.VMEM((CH, D), jnp.float32),
        pltpu.VMEM((CH, L), jnp.float32),
        pltpu.VMEM_SHARED((R, D), jnp.float32),
        pltpu.VMEM_SHARED((R, L), jnp.float32),
        pltpu.SemaphoreType.DMA,
    ],
)
def _edge_kernel(edge, slotu_h, sloti_h, uemb, iemb,
                 aggi, aggu, degi, degu,
                 table, keyb, payb, slotb, rows, ones, acc, deg, sem):
    c = lax.axis_index("c")
    s = lax.axis_index("s")
    NCH = N_EDGES // CH

    @pl.when(c == 0)
    def _():
        pltpu.sync_copy(slotu_h, table)

    @pl.when(c == 1)
    def _():
        pltpu.sync_copy(sloti_h, table)

    zf = jnp.zeros((L,), jnp.float32)

    def zero_body(i, carry):
        for j in range(D // L):
            rows[i, pl.ds(j * L, L)] = zf
        ones[i, :] = zf
        return carry
    lax.fori_loop(0, CH, zero_body, 0)

    def initacc_body(i, carry):
        cb = s + i * NSUB
        pltpu.sync_copy(rows, acc.at[pl.ds(cb * CH, CH), :])
        pltpu.sync_copy(ones, deg.at[pl.ds(cb * CH, CH), :])
        return carry
    lax.fori_loop(0, (RCH - s + NSUB - 1) // NSUB, initacc_body, 0)

    onev = jnp.ones((L,), jnp.float32)

    def ones_body(i, carry):
        ones[i, :] = onev
        return carry
    lax.fori_loop(0, CH, ones_body, 0)

    plsc.subcore_barrier()

    def edge_body(i, carry):
        ch = s + i * NSUB
        off = ch * CH

        @pl.when(c == 0)
        def _():
            pltpu.sync_copy(edge.at[0, pl.ds(off, CH)], keyb)
            pltpu.sync_copy(edge.at[1, pl.ds(off, CH)], payb)

        @pl.when(c == 1)
        def _():
            pltpu.sync_copy(edge.at[1, pl.ds(off, CH)], keyb)
            pltpu.sync_copy(edge.at[0, pl.ds(off, CH)], payb)

        for k in range(CH // L):
            kv = keyb[pl.ds(k * L, L)]
            wv = plsc.load_gather(table, [lax.shift_right_logical(kv, 1)])
            slot = jnp.where((kv & 1) == 1,
                             lax.shift_right_logical(wv, 16),
                             wv & 0xFFFF)
            slotb[pl.ds(k * L, L)] = slot

        @pl.when(c == 0)
        def _():
            pltpu.async_copy(iemb.at[payb], rows, sem).wait()

        @pl.when(c == 1)
        def _():
            pltpu.async_copy(uemb.at[payb], rows, sem).wait()

        pltpu.sync_copy(rows, acc.at[slotb], add=True)
        pltpu.sync_copy(ones, deg.at[slotb], add=True)
        return carry

    lax.fori_loop(0, (NCH - s + NSUB - 1) // NSUB, edge_body, 0)

    plsc.subcore_barrier()

    def drain_body(i, carry):
        cb = s + i * NSUB
        sl = pl.ds(cb * CH, CH)

        @pl.when(c == 0)
        def _():
            pltpu.sync_copy(acc.at[sl, :], aggi.at[sl, :])
            pltpu.sync_copy(deg.at[sl, :], degi.at[sl, :])

        @pl.when(c == 1)
        def _():
            pltpu.sync_copy(acc.at[sl, :], aggu.at[sl, :])
            pltpu.sync_copy(deg.at[sl, :], degu.at[sl, :])
        return carry
    lax.fori_loop(0, (RCH - s + NSUB - 1) // NSUB, drain_body, 0)


@functools.partial(
    pl.kernel,
    out_type=(
        jax.ShapeDtypeStruct((B, D), jnp.float32),
        jax.ShapeDtypeStruct((B, D), jnp.float32),
        jax.ShapeDtypeStruct((B, D), jnp.float32),
        jax.ShapeDtypeStruct((B, D), jnp.float32),
        jax.ShapeDtypeStruct((B,), jnp.float32),
        jax.ShapeDtypeStruct((B,), jnp.float32),
    ),
    mesh=_mesh,
    compiler_params=_sc_params,
    scratch_types=[
        pltpu.VMEM((2 * Q_PER_TILE,), jnp.int32),
        pltpu.VMEM((Q_PER_TILE,), jnp.int32),
        pltpu.VMEM((Q_PER_TILE,), jnp.int32),
        pltpu.VMEM((Q_PER_TILE, D), jnp.float32),
        pltpu.VMEM((Q_PER_TILE, D), jnp.float32),
        pltpu.VMEM((Q_PER_TILE, L), jnp.float32),
        pltpu.VMEM((N_USER,), jnp.float32),
        pltpu.VMEM((Q_PER_TILE,), jnp.float32),
        pltpu.SemaphoreType.DMA,
    ],
)
def _query_kernel(x_hbm, repu_h, repi_h, uemb, iemb,
                  aggi_h, aggu_h, degi_h, degu_h, ubias_h, ibias_h,
                  ueq, ieq, gcni, gcnu, ubq_o, ibq_o,
                  xv, idsv, repv, erows, arows, drows, btab, bq, sem):
    c = lax.axis_index("c")
    s = lax.axis_index("s")
    w = s * 2 + c
    qb = w * Q_PER_TILE

    pltpu.sync_copy(x_hbm.at[pl.ds(2 * qb, 2 * Q_PER_TILE)], xv)

    def _side(rep_h, emb_h, agg_h, deg_h, bias_h, emb_o, gcn_o, b_o, col):
        pltpu.sync_copy(rep_h.at[pl.ds(qb, Q_PER_TILE)], repv)
        pltpu.sync_copy(bias_h, btab)

        def ext_body(q, carry):
            qv = q * L + _iota16()
            idsv[pl.ds(q * L, L)] = plsc.load_gather(xv, [qv * 2 + col])
            return carry
        lax.fori_loop(0, Q_PER_TILE // L, ext_body, 0)

        pltpu.async_copy(emb_h.at[idsv], erows, sem).wait()
        pltpu.sync_copy(erows, emb_o.at[pl.ds(qb, Q_PER_TILE), :])

        pltpu.async_copy(agg_h.at[repv], arows, sem).wait()
        pltpu.async_copy(deg_h.at[repv], drows, sem).wait()

        def bias_body(q, carry):
            kv = idsv[pl.ds(q * L, L)]
            bq[pl.ds(q * L, L)] = plsc.load_gather(btab, [kv])
            return carry
        lax.fori_loop(0, Q_PER_TILE // L, bias_body, 0)
        pltpu.sync_copy(bq, b_o.at[pl.ds(qb, Q_PER_TILE)])

        def norm_body(r, carry):
            dv = drows[r, :]
            inv = 1.0 / (dv + 1.0)
            for j in range(D // L):
                sl = pl.ds(j * L, L)
                arows[r, sl] = arows[r, sl] * inv
            return carry
        lax.fori_loop(0, Q_PER_TILE, norm_body, 0)
        pltpu.sync_copy(arows, gcn_o.at[pl.ds(qb, Q_PER_TILE), :])

    @pl.when(c == 0)
    def _():
        _side(repu_h, uemb, aggi_h, degi_h, ubias_h, ueq, gcni, ubq_o, 0)

    @pl.when(c == 1)
    def _():
        _side(repi_h, iemb, aggu_h, degu_h, ibias_h, ieq, gcnu, ibq_o, 1)


BM = 2048


def _dense_body(ue_r, ie_r, gi_r, gu_r, bq_r,
                wgu_r, bgu_r, wgi_r, bgi_r,
                w1_r, b1_r, w2_r, b2_r, w3_r, b3_r, out_r):
    f32 = jnp.float32
    dn = (((1,), (1,)), ((), ()))
    ue = ue_r[...]
    ie = ie_r[...]
    gu = lax.dot_general(gu_r[...], wgu_r[...], dn,
                         preferred_element_type=f32) + bgu_r[...]
    gu = jnp.maximum(gu, 0.0)
    gi = lax.dot_general(gi_r[...], wgi_r[...], dn,
                         preferred_element_type=f32) + bgi_r[...]
    gi = jnp.maximum(gi, 0.0)
    xcat = jnp.concatenate([ue * ie, ue * gi, gu * ie, gu * gi], axis=-1)
    x1 = jnp.tanh(lax.dot_general(xcat, w1_r[...], dn,
                                  preferred_element_type=f32) + b1_r[...])
    x2 = jnp.tanh(lax.dot_general(x1, w2_r[...], dn,
                                  preferred_element_type=f32) + b2_r[...])
    x3 = jnp.sum(x2 * w3_r[...], axis=1) + b3_r[0, 0]
    out_r[...] = x3 + bq_r[...]


def _dense(ue, ie, gi, gu, bq, W_gu, b_gu, W_gi, b_gi, W1, b1, W2, b2, W3, b3):
    n_blocks = B // BM
    feat = pl.BlockSpec((BM, D), lambda i: (i, 0))
    vec = pl.BlockSpec((BM,), lambda i: (i,))

    def full(shape):
        return pl.BlockSpec(shape, lambda i: tuple(0 for _ in shape))

    return pl.pallas_call(
        _dense_body,
        grid=(n_blocks,),
        in_specs=[feat, feat, feat, feat, vec,
                  full((D, D)), full((1, D)), full((D, D)), full((1, D)),
                  full((2 * D, 4 * D)), full((1, 2 * D)),
                  full((D, 2 * D)), full((1, D)),
                  full((1, D)), full((1, 1))],
        out_specs=vec,
        out_shape=jax.ShapeDtypeStruct((B,), jnp.float32),
    )(ue, ie, gi, gu, bq, W_gu, b_gu, W_gi, b_gi, W1, b1, W2, b2, W3, b3)


def kernel(x, edge_index, user_embedding, item_embedding,
           W_gu, b_gu, W_gi, b_gi, W1, b1, W2, b2, W3, b3,
           user_bias, item_bias):
    x_flat = x.astype(jnp.int32).reshape(-1)
    edge_index = edge_index.astype(jnp.int32)

    slotu, sloti, repu, repi = _slot_kernel(x_flat)
    aggi, aggu, degi, degu = _edge_kernel(
        edge_index, slotu, sloti, user_embedding, item_embedding)
    ueq, ieq, gcni, gcnu, ubq, ibq = _query_kernel(
        x_flat, repu, repi, user_embedding, item_embedding,
        aggi, aggu, degi, degu,
        user_bias.reshape(-1), item_bias.reshape(-1))

    return _dense(ueq, ieq, gcni, gcnu, ubq + ibq,
                  W_gu, b_gu.reshape(1, D), W_gi, b_gi.reshape(1, D),
                  W1, b1.reshape(1, 2 * D), W2, b2.reshape(1, D),
                  W3, b3.reshape(1, 1))
